# Initial kernel scaffold; baseline (speedup 1.0000x reference)
import jax
import jax.numpy as jnp
from jax.experimental import pallas as pl


def kernel(input, table):
    raise NotImplementedError("write your pallas kernel here")


# speedup vs baseline: 1.8074x; 1.8074x over previous
import functools

import jax
import jax.numpy as jnp
from jax import lax
from jax.experimental import pallas as pl
from jax.experimental.pallas import tpu as pltpu
from jax.experimental.pallas import tpu_sc as plsc

_D = 32
_ROWS_PER_DMA = 128


@functools.cache
def _make_lookup(B):
    info = plsc.get_sparse_core_info()
    nc = info.num_cores
    nw = nc * info.num_subcores
    b_per_w = B // nw
    n_dma = b_per_w // _ROWS_PER_DMA
    mesh = plsc.VectorSubcoreMesh(core_axis_name="c", subcore_axis_name="s")

    @functools.partial(
        pl.kernel,
        mesh=mesh,
        compiler_params=pltpu.CompilerParams(use_tc_tiling_on_sc=False),
        out_type=jax.ShapeDtypeStruct((B, _D), jnp.float32),
        scratch_types=[
            pltpu.VMEM((n_dma, _ROWS_PER_DMA), jnp.int32),
            pltpu.VMEM((_ROWS_PER_DMA, _D), jnp.float32),
            pltpu.SemaphoreType.DMA,
        ],
    )
    def k(table_hbm, idx_hbm, out_hbm, idx_v, rows_v, sem):
        wid = lax.axis_index("s") * nc + lax.axis_index("c")
        pltpu.sync_copy(idx_hbm.at[wid], idx_v)
        base = pl.multiple_of(wid * b_per_w, _ROWS_PER_DMA)

        def body(j, carry):
            pltpu.async_copy(table_hbm.at[idx_v.at[j]], rows_v, sem).wait()
            pltpu.sync_copy(
                rows_v, out_hbm.at[pl.ds(base + j * _ROWS_PER_DMA, _ROWS_PER_DMA)]
            )
            return carry

        lax.fori_loop(0, n_dma, body, 0)

    return k


def kernel(input, table):
    B = input.size
    info = plsc.get_sparse_core_info()
    nw = info.num_cores * info.num_subcores
    idx3d = input.reshape(nw, B // (nw * _ROWS_PER_DMA), _ROWS_PER_DMA)
    out = _make_lookup(B)(table, idx3d)
    return out.reshape(input.shape + (_D,))


# speedup vs baseline: 1.8872x; 1.0442x over previous
import functools

import jax
import jax.numpy as jnp
from jax import lax
from jax.experimental import pallas as pl
from jax.experimental.pallas import tpu as pltpu
from jax.experimental.pallas import tpu_sc as plsc

_D = 32
_CH = 640
_NBUF = 4
_LAG = 2


@functools.cache
def _make_lookup(B):
    info = plsc.get_sparse_core_info()
    nc = info.num_cores
    nw = nc * info.num_subcores
    b_per_w = B // nw
    n_dma = b_per_w // _CH
    assert n_dma * _CH == b_per_w and n_dma >= _NBUF >= _LAG
    mesh = plsc.VectorSubcoreMesh(core_axis_name="c", subcore_axis_name="s")

    @functools.partial(
        pl.kernel,
        mesh=mesh,
        compiler_params=pltpu.CompilerParams(use_tc_tiling_on_sc=False),
        out_type=jax.ShapeDtypeStruct((B, _D), jnp.float32),
        scratch_types=[
            pltpu.VMEM((n_dma, _CH), jnp.int32),
            pltpu.VMEM((_NBUF, _CH, _D), jnp.float32),
            pltpu.SemaphoreType.DMA((_NBUF,)),
            pltpu.SemaphoreType.DMA((_NBUF,)),
        ],
    )
    def k(table_hbm, idx_hbm, out_hbm, idx_v, rows_v, gsem, ssem):
        wid = lax.axis_index("s") * nc + lax.axis_index("c")
        pltpu.sync_copy(idx_hbm.at[wid], idx_v)
        base = pl.multiple_of(wid * b_per_w, _CH)

        def gather(j):
            return pltpu.async_copy(
                table_hbm.at[idx_v.at[j]], rows_v.at[j % _NBUF], gsem.at[j % _NBUF]
            )

        def store(j):
            return pltpu.async_copy(
                rows_v.at[j % _NBUF],
                out_hbm.at[pl.ds(base + j * _CH, _CH)],
                ssem.at[j % _NBUF],
            )

        gathers = {}
        stores = {}
        for j in range(_LAG):
            gathers[j] = gather(j)
        for j in range(n_dma):
            gathers.pop(j).wait()
            stores[j] = store(j)
            jn = j + _LAG
            if jn < n_dma:
                if jn >= _NBUF:
                    stores.pop(jn - _NBUF).wait()
                gathers[jn] = gather(jn)
        for j in sorted(stores):
            stores.pop(j).wait()

    return k


def kernel(input, table):
    B = input.size
    info = plsc.get_sparse_core_info()
    nw = info.num_cores * info.num_subcores
    idx3d = input.reshape(nw, B // (nw * _CH), _CH)
    out = _make_lookup(B)(table, idx3d)
    return out.reshape(input.shape + (_D,))


# speedup vs baseline: 1.8893x; 1.0011x over previous
import functools

import jax
import jax.numpy as jnp
from jax import lax
from jax.experimental import pallas as pl
from jax.experimental.pallas import tpu as pltpu
from jax.experimental.pallas import tpu_sc as plsc

_D = 32
_CH = 256
_NBUF = 12
_LAG = 8


@functools.cache
def _make_lookup(B):
    info = plsc.get_sparse_core_info()
    nc = info.num_cores
    nw = nc * info.num_subcores
    b_per_w = B // nw
    n_dma = b_per_w // _CH
    assert n_dma * _CH == b_per_w and n_dma >= _NBUF >= _LAG
    mesh = plsc.VectorSubcoreMesh(core_axis_name="c", subcore_axis_name="s")

    @functools.partial(
        pl.kernel,
        mesh=mesh,
        compiler_params=pltpu.CompilerParams(use_tc_tiling_on_sc=False),
        out_type=jax.ShapeDtypeStruct((B, _D), jnp.float32),
        scratch_types=[
            pltpu.VMEM((n_dma, _CH), jnp.int32),
            pltpu.VMEM((_NBUF, _CH, _D), jnp.float32),
            pltpu.SemaphoreType.DMA((_NBUF,)),
            pltpu.SemaphoreType.DMA((_NBUF,)),
        ],
    )
    def k(table_hbm, idx_hbm, out_hbm, idx_v, rows_v, gsem, ssem):
        wid = lax.axis_index("s") * nc + lax.axis_index("c")
        pltpu.sync_copy(idx_hbm.at[wid], idx_v)
        base = pl.multiple_of(wid * b_per_w, _CH)

        def gather(j):
            return pltpu.async_copy(
                table_hbm.at[idx_v.at[j]], rows_v.at[j % _NBUF], gsem.at[j % _NBUF]
            )

        def store(j):
            return pltpu.async_copy(
                rows_v.at[j % _NBUF],
                out_hbm.at[pl.ds(base + j * _CH, _CH)],
                ssem.at[j % _NBUF],
            )

        gathers = {}
        stores = {}
        for j in range(_LAG):
            gathers[j] = gather(j)
        for j in range(n_dma):
            gathers.pop(j).wait()
            stores[j] = store(j)
            jn = j + _LAG
            if jn < n_dma:
                if jn >= _NBUF:
                    stores.pop(jn - _NBUF).wait()
                gathers[jn] = gather(jn)
        for j in sorted(stores):
            stores.pop(j).wait()

    return k


def kernel(input, table):
    B = input.size
    info = plsc.get_sparse_core_info()
    nw = info.num_cores * info.num_subcores
    idx3d = input.reshape(nw, B // (nw * _CH), _CH)
    out = _make_lookup(B)(table, idx3d)
    return out.reshape(input.shape + (_D,))
